# Initial kernel scaffold; baseline (speedup 1.0000x reference)
import functools

import jax
import jax.numpy as jnp
from jax import lax
from jax.experimental import pallas as pl
from jax.experimental.pallas import tpu as pltpu

RES = 0.5
DIM = 23.5
NPTS = 48
YZ = NPTS * NPTS
ORIGIN = -DIM / 2.0
E2 = 0.1353352832366127


def _tc_body(coords_ref, types_t_ref, radii_ref, out_ref):
    cx = coords_ref[0, 0, :]
    cy = coords_ref[0, 1, :]
    cz = coords_ref[0, 2, :]
    r = radii_ref[0, 0, :]
    r2 = r * r
    inv_r2 = 1.0 / r2
    inv_r = 1.0 / r
    c1 = (4.0 * E2) * inv_r2
    c2 = (12.0 * E2) * inv_r
    c3 = jnp.full_like(r, 9.0 * E2)

    lin = lax.broadcasted_iota(jnp.int32, (1, YZ), 1)
    ay = ORIGIN + RES * (lin // NPTS).astype(jnp.float32)
    az = ORIGIN + RES * (lin % NPTS).astype(jnp.float32)
    dy = cy[:, None] - ay
    dz = cz[:, None] - az
    dyz2 = dy * dy + dz * dz
    ayz = jnp.exp(-2.0 * inv_r2[:, None] * dyz2)

    xi = lax.broadcasted_iota(jnp.float32, (1, NPTS), 1)
    ax = ORIGIN + RES * xi
    dx = cx[:, None] - ax
    dx2 = dx * dx
    axg = jnp.exp(-2.0 * inv_r2[:, None] * dx2)

    types_t = types_t_ref[0]
    r2b = r2[:, None]
    q225 = 2.25 * r2b

    def per_x(x, _):
        d2 = dx2[:, x][:, None] + dyz2
        gauss = axg[:, x][:, None] * ayz
        d = jnp.sqrt(d2)
        q = (c1[:, None] * d2 - c2[:, None] * d) + c3[:, None]
        dens = jnp.where(d2 < r2b, gauss, jnp.where(d2 < q225, q, 0.0))
        mm = jax.lax.dot_general(
            types_t, dens, (((1,), (0,)), ((), ())),
            preferred_element_type=jnp.float32)
        out_ref[0, :, x, :] = mm
        return ()

    lax.fori_loop(0, NPTS, per_x, (), unroll=4)


@jax.jit
def kernel(coords, types, radii):
    B, N, _ = coords.shape
    T = types.shape[-1]
    coords_t = jnp.swapaxes(coords, 1, 2)
    types_t = jnp.swapaxes(types, 1, 2)
    radii_r = radii[:, None, :]

    out = pl.pallas_call(
        _tc_body,
        grid=(B,),
        in_specs=[
            pl.BlockSpec((1, 3, N), lambda b: (b, 0, 0)),
            pl.BlockSpec((1, T, N), lambda b: (b, 0, 0)),
            pl.BlockSpec((1, 1, N), lambda b: (b, 0, 0)),
        ],
        out_specs=pl.BlockSpec((1, T, NPTS, YZ), lambda b: (b, 0, 0, 0)),
        out_shape=jax.ShapeDtypeStruct((B, T, NPTS, YZ), jnp.float32),
    )(coords_t, types_t, radii_r)
    return out.reshape(B, T, NPTS, NPTS, NPTS)


# speedup vs baseline: 3.1021x; 3.1021x over previous
import jax
import jax.numpy as jnp
from jax import lax
from jax.experimental import pallas as pl
from jax.experimental.pallas import tpu as pltpu

RES = 0.5
DIM = 23.5
NPTS = 48
YZ = NPTS * NPTS
ORIGIN = -DIM / 2.0
E2 = 0.1353352832366127


XB = 8


def _tc_body(coords_ref, types_t_ref, radii_ref, out_ref, dyz2_ref, ayz_ref):
    xs = pl.program_id(1)
    cx = coords_ref[0, 0, :]
    r = radii_ref[0, 0, :]
    r2 = r * r
    inv_r2 = 1.0 / r2

    @pl.when(xs == 0)
    def _():
        cy = coords_ref[0, 1, :]
        cz = coords_ref[0, 2, :]
        lin = lax.broadcasted_iota(jnp.int32, (1, YZ), 1)
        ay = ORIGIN + RES * (lin // NPTS).astype(jnp.float32)
        az = ORIGIN + RES * (lin % NPTS).astype(jnp.float32)
        dy = cy[:, None] - ay
        dz = cz[:, None] - az
        d2 = dy * dy + dz * dz
        dyz2_ref[...] = d2
        ayz_ref[...] = jnp.exp(-2.0 * inv_r2[:, None] * d2)

    inv_r = 1.0 / r
    c1 = (4.0 * E2) * inv_r2
    c2 = (12.0 * E2) * inv_r
    c3 = 9.0 * E2
    r2b = r2[:, None]
    xf = xs.astype(jnp.float32)
    dyz2 = dyz2_ref[...]
    ayz = ayz_ref[...]
    types_t = types_t_ref[0]

    for j in range(XB):
        ax = ORIGIN + RES * (xf * XB + j)
        dx = cx - ax
        dx2 = dx * dx
        axg = jnp.exp(-2.0 * inv_r2 * dx2)
        d2 = dx2[:, None] + dyz2
        gauss = axg[:, None] * ayz
        d = jnp.sqrt(d2)
        q = (c1[:, None] * d2 - c2[:, None] * d) + c3
        dens = jnp.where(d2 < r2b, gauss, jnp.where(d2 < 2.25 * r2b, q, 0.0))
        mm = jax.lax.dot_general(
            types_t, dens, (((1,), (0,)), ((), ())),
            preferred_element_type=jnp.float32)
        out_ref[0, :, j, :] = mm


@jax.jit
def kernel(coords, types, radii):
    B, N, _ = coords.shape
    T = types.shape[-1]
    coords_t = jnp.swapaxes(coords, 1, 2)
    types_t = jnp.swapaxes(types, 1, 2)
    radii_r = radii[:, None, :]

    out = pl.pallas_call(
        _tc_body,
        grid=(B, NPTS // XB),
        in_specs=[
            pl.BlockSpec((1, 3, N), lambda b, x: (b, 0, 0)),
            pl.BlockSpec((1, T, N), lambda b, x: (b, 0, 0)),
            pl.BlockSpec((1, 1, N), lambda b, x: (b, 0, 0)),
        ],
        out_specs=pl.BlockSpec((1, T, XB, YZ), lambda b, x: (b, 0, x, 0)),
        out_shape=jax.ShapeDtypeStruct((B, T, NPTS, YZ), jnp.float32),
        scratch_shapes=[
            pltpu.VMEM((N, YZ), jnp.float32),
            pltpu.VMEM((N, YZ), jnp.float32),
        ],
    )(coords_t, types_t, radii_r)
    return out.reshape(B, T, NPTS, NPTS, NPTS)
